# Initial kernel scaffold; baseline (speedup 1.0000x reference)
import jax
import jax.numpy as jnp
from jax.experimental import pallas as pl


def kernel(x_embed, prompt):
    raise NotImplementedError("write your pallas kernel here")


# speedup vs baseline: 1.1298x; 1.1298x over previous
import jax
import jax.numpy as jnp
from jax import lax
from jax.experimental import pallas as pl
from jax.experimental.pallas import tpu as pltpu

POOL = 1024
LEN = 16
K = 8
B, T, H = 4, 8192, 1024

TBLK = 128
NT = T // TBLK
PBLK = 128
NP = POOL // PBLK
OUT_T = K * LEN + T


def _copy_mean_body(x_ref, out_ref, sum_ref):
    t = pl.program_id(0)
    xb = x_ref[...]
    out_ref[...] = xb

    @pl.when(t == 0)
    def _():
        sum_ref[...] = jnp.zeros_like(sum_ref)

    sum_ref[...] += jnp.sum(xb, axis=1)


def _pmean_body(p_ref, pm_ref):
    pm_ref[...] = jnp.mean(p_ref[...], axis=1)


def _sim_topk_body(xs_ref, pm_ref, sim_ref, idx_ref, rs_ref):
    xm = xs_ref[...] * (1.0 / T)
    xss = jnp.sum(xm * xm, axis=1, keepdims=True)
    xn = xm * lax.rsqrt(jnp.maximum(xss, 1e-12))
    pm = pm_ref[...]
    pss = jnp.sum(pm * pm, axis=1, keepdims=True)
    pn = pm * lax.rsqrt(jnp.maximum(pss, 1e-12))
    sim = lax.dot_general(
        xn, pn, (((1,), (1,)), ((), ())), preferred_element_type=jnp.float32
    )
    sim_ref[...] = sim

    iota = lax.broadcasted_iota(jnp.int32, (B, POOL), 1)
    cur = sim
    total = jnp.float32(0.0)
    cols = []
    for _ in range(K):
        m = jnp.max(cur, axis=1, keepdims=True)
        cand = jnp.where(cur == m, iota, POOL)
        i = jnp.min(cand, axis=1, keepdims=True)
        cols.append(i)
        total += jnp.sum(m)
        cur = jnp.where(iota == i, -jnp.inf, cur)
    idx_ref[...] = jnp.concatenate(cols, axis=1)
    rs_ref[0, 0] = total * (1.0 / B)


def _gather_body(idx_ref, p_ref, big_ref, out_ref):
    del idx_ref, big_ref
    out_ref[...] = p_ref[...]


def kernel(x_embed, prompt):
    big0, x_sum = pl.pallas_call(
        _copy_mean_body,
        grid=(NT,),
        in_specs=[pl.BlockSpec((B, TBLK, H), lambda t: (0, t, 0))],
        out_specs=[
            pl.BlockSpec((B, TBLK, H), lambda t: (0, t + K * LEN // TBLK, 0)),
            pl.BlockSpec((B, H), lambda t: (0, 0)),
        ],
        out_shape=[
            jax.ShapeDtypeStruct((B, OUT_T, H), jnp.float32),
            jax.ShapeDtypeStruct((B, H), jnp.float32),
        ],
    )(x_embed)

    pm = pl.pallas_call(
        _pmean_body,
        grid=(NP,),
        in_specs=[pl.BlockSpec((PBLK, LEN, H), lambda p: (p, 0, 0))],
        out_specs=pl.BlockSpec((PBLK, H), lambda p: (p, 0)),
        out_shape=jax.ShapeDtypeStruct((POOL, H), jnp.float32),
    )(prompt)

    sim, idx, rs = pl.pallas_call(
        _sim_topk_body,
        out_specs=[
            pl.BlockSpec(memory_space=pltpu.VMEM),
            pl.BlockSpec(memory_space=pltpu.VMEM),
            pl.BlockSpec(memory_space=pltpu.SMEM),
        ],
        out_shape=[
            jax.ShapeDtypeStruct((B, POOL), jnp.float32),
            jax.ShapeDtypeStruct((B, K), jnp.int32),
            jax.ShapeDtypeStruct((1, 1), jnp.float32),
        ],
    )(x_sum, pm)

    big = pl.pallas_call(
        _gather_body,
        grid_spec=pltpu.PrefetchScalarGridSpec(
            num_scalar_prefetch=1,
            grid=(B, K),
            in_specs=[
                pl.BlockSpec((1, LEN, H), lambda b, k, idx_p: (idx_p[b, k], 0, 0)),
                pl.BlockSpec((1, LEN, H), lambda b, k, idx_p: (b, k, 0)),
            ],
            out_specs=pl.BlockSpec((1, LEN, H), lambda b, k, idx_p: (b, k, 0)),
        ),
        out_shape=jax.ShapeDtypeStruct((B, OUT_T, H), jnp.float32),
        input_output_aliases={2: 0},
    )(idx, prompt, big0)

    return big, rs[0, 0], sim, idx
